# Initial kernel scaffold; baseline (speedup 1.0000x reference)
import jax
import jax.numpy as jnp
from jax.experimental import pallas as pl


def kernel(inputs, weight):
    raise NotImplementedError("write your pallas kernel here")


# speedup vs baseline: 1.4907x; 1.4907x over previous
import functools

import jax
import jax.numpy as jnp
from jax import lax
from jax.experimental import pallas as pl
from jax.experimental.pallas import tpu as pltpu
from jax.experimental.pallas import tpu_sc as plsc

_EMBED_DIMS = 32
_NUM_WORKERS = 32
_CHUNK = 1024


def _embedding_gather(num_idx):
    b_per_w = num_idx // _NUM_WORKERS
    num_chunks = b_per_w // _CHUNK
    mesh = plsc.VectorSubcoreMesh(core_axis_name="c", subcore_axis_name="s")

    @functools.partial(
        pl.kernel,
        mesh=mesh,
        out_type=jax.ShapeDtypeStruct((num_idx, _EMBED_DIMS), jnp.float32),
        scratch_types=[
            pltpu.VMEM((_CHUNK,), jnp.int32),
            pltpu.VMEM((_CHUNK, _EMBED_DIMS), jnp.float32),
            pltpu.SemaphoreType.DMA,
        ],
        compiler_params=pltpu.CompilerParams(use_tc_tiling_on_sc=False),
    )
    def k(idx_hbm, table_hbm, out_hbm, idx_v, rows_v, sem):
        wid = lax.axis_index("s") * 2 + lax.axis_index("c")
        base = wid * b_per_w

        def body(i, carry):
            off = base + i * _CHUNK
            pltpu.sync_copy(idx_hbm.at[pl.ds(off, _CHUNK)], idx_v)
            pltpu.async_copy(table_hbm.at[idx_v], rows_v, sem).wait()
            pltpu.sync_copy(rows_v, out_hbm.at[pl.ds(off, _CHUNK)])
            return carry

        lax.fori_loop(0, num_chunks, body, 0)

    return k


def kernel(inputs, weight):
    batch, hist = inputs.shape
    flat_idx = inputs.reshape(batch * hist)
    out = _embedding_gather(batch * hist)(flat_idx, weight)
    return out.reshape(batch, hist, _EMBED_DIMS)


# speedup vs baseline: 1.5134x; 1.0152x over previous
import functools

import jax
import jax.numpy as jnp
from jax import lax
from jax.experimental import pallas as pl
from jax.experimental.pallas import tpu as pltpu
from jax.experimental.pallas import tpu_sc as plsc

_EMBED_DIMS = 32
_NUM_WORKERS = 32
_CHUNK = 1024
_NBUF = 3


def _embedding_gather(num_idx):
    b_per_w = num_idx // _NUM_WORKERS
    num_chunks = b_per_w // _CHUNK
    mesh = plsc.VectorSubcoreMesh(core_axis_name="c", subcore_axis_name="s")

    scratch = (
        [pltpu.VMEM((num_chunks, _CHUNK), jnp.int32)]
        + [pltpu.VMEM((_CHUNK, _EMBED_DIMS), jnp.float32) for _ in range(_NBUF)]
        + [pltpu.SemaphoreType.DMA for _ in range(2 * _NBUF)]
    )

    @functools.partial(
        pl.kernel,
        mesh=mesh,
        out_type=jax.ShapeDtypeStruct((num_idx, _EMBED_DIMS), jnp.float32),
        scratch_types=scratch,
        compiler_params=pltpu.CompilerParams(use_tc_tiling_on_sc=False),
    )
    def k(idx_hbm, table_hbm, out_hbm, idx_v, *bufs_and_sems):
        rows = bufs_and_sems[:_NBUF]
        g_sem = bufs_and_sems[_NBUF:2 * _NBUF]
        s_sem = bufs_and_sems[2 * _NBUF:]

        wid = lax.axis_index("s") * 2 + lax.axis_index("c")
        chunk0 = wid * num_chunks

        pltpu.sync_copy(idx_hbm.at[pl.ds(chunk0, num_chunks)], idx_v)

        def start_gather(i):
            b = i % _NBUF
            return pltpu.async_copy(table_hbm.at[idx_v.at[i]], rows[b], g_sem[b])

        def start_store(i):
            b = i % _NBUF
            return pltpu.async_copy(
                rows[b], out_hbm.at[pl.ds((chunk0 + i) * _CHUNK, _CHUNK)], s_sem[b]
            )

        gathers = [None] * num_chunks
        stores = [None] * num_chunks
        for i in range(min(_NBUF - 1, num_chunks)):
            gathers[i] = start_gather(i)
        for i in range(num_chunks):
            if i > 0:
                stores[i - 1].wait()
            j = i + _NBUF - 1
            if j < num_chunks:
                gathers[j] = start_gather(j)
            gathers[i].wait()
            stores[i] = start_store(i)
        stores[num_chunks - 1].wait()

    return k


def kernel(inputs, weight):
    batch, hist = inputs.shape
    num_idx = batch * hist
    b_per_w = num_idx // _NUM_WORKERS
    flat_idx = inputs.reshape(num_idx // _CHUNK, _CHUNK)
    out = _embedding_gather(num_idx)(flat_idx, weight)
    return out.reshape(batch, hist, _EMBED_DIMS)
